# baseline (speedup 1.0000x reference)
import functools

import jax
import jax.numpy as jnp
from jax import lax
from jax.experimental import pallas as pl
from jax.experimental.pallas import tpu as pltpu
from jax.experimental.pallas import tpu_sc as plsc

N = 10000
E = 320000
D = 128
N_SITES = 1024
NC = 2
NSUB = 16
NW = NC * NSUB
EPT = E // NW
L = 16

SLOT_PAD = 1152
DUMMY = N_SITES
SPT = N_SITES // NSUB
CAP = EPT + SPT + 512

_mesh = plsc.VectorSubcoreMesh(core_axis_name="c", subcore_axis_name="s")


def _iota16():
    return lax.broadcasted_iota(jnp.int32, (L,), 0)


@functools.partial(
    pl.kernel,
    mesh=_mesh,
    out_type=[
        jax.ShapeDtypeStruct((NW, N), jnp.float32),
        jax.ShapeDtypeStruct((N,), jnp.int32),
        jax.ShapeDtypeStruct((1, N_SITES), jnp.int32),
    ],
    scratch_types=[
        pltpu.VMEM((EPT,), jnp.int32),
        pltpu.VMEM((N,), jnp.float32),
        pltpu.VMEM((N,), jnp.int32),
        pltpu.VMEM((N_SITES,), jnp.int32),
        pltpu.VMEM((N_SITES,), jnp.int32),
    ],
    compiler_params=pltpu.CompilerParams(needs_layout_passes=False),
)
def _sc_deg_pos(dst_hbm, ni_hbm, deg_out, pos_out, ps_out,
                dst_v, hist_v, pos_v, ni_v, ps_v):
    c = lax.axis_index("c")
    s = lax.axis_index("s")
    wid = s * NC + c
    ones = jnp.ones((L,), jnp.float32)
    zeros = jnp.zeros((L,), jnp.float32)
    iota = _iota16()

    pltpu.sync_copy(dst_hbm.at[pl.ds(wid * EPT, EPT)], dst_v)

    def _zero(k, _):
        hist_v[pl.ds(k * L, L)] = zeros
        return 0
    lax.fori_loop(0, N // L, _zero, 0)

    def _hist(k, _):
        d = dst_v[pl.ds(k * L, L)]
        plsc.addupdate_scatter(hist_v, [d], ones)
        return 0
    lax.fori_loop(0, EPT // L, _hist, 0)

    pltpu.sync_copy(hist_v, deg_out.at[wid])

    @pl.when(jnp.logical_and(c == 0, s == 0))
    def _tile0():
        pltpu.sync_copy(ni_hbm, ni_v)
        neg = jnp.full((L,), -1, jnp.int32)

        def _init(k, _):
            pos_v[pl.ds(k * L, L)] = neg
            return 0
        lax.fori_loop(0, N // L, _init, 0)

        def _scat(k, _):
            idx = ni_v[pl.ds(k * L, L)]
            plsc.store_scatter(pos_v, [idx], iota + k * L)
            return 0
        lax.fori_loop(0, N_SITES // L, _scat, 0)

        def _gath(k, _):
            idx = ni_v[pl.ds(k * L, L)]
            ps_v[pl.ds(k * L, L)] = plsc.load_gather(pos_v, [idx])
            return 0
        lax.fori_loop(0, N_SITES // L, _gath, 0)

        pltpu.sync_copy(pos_v, pos_out)
        pltpu.sync_copy(ps_v, ps_out.at[0])


def _tc_y_body(x_ref, w_ref, deg_ref, y_ref, dinv_ref):
    deg = jnp.sum(deg_ref[...], axis=0) + 1.0
    dinv = lax.rsqrt(deg)
    xw = jnp.dot(x_ref[...], w_ref[...], preferred_element_type=jnp.float32)
    y_ref[...] = xw * dinv[:, None]
    dinv_ref[...] = dinv[None, :]


def _tc_y(x, w_conv, deg):
    return pl.pallas_call(
        _tc_y_body,
        out_shape=[
            jax.ShapeDtypeStruct((N, D), jnp.float32),
            jax.ShapeDtypeStruct((1, N), jnp.float32),
        ],
    )(x, w_conv, deg)


@functools.partial(
    pl.kernel,
    mesh=_mesh,
    out_type=[
        jax.ShapeDtypeStruct((NC, SLOT_PAD, D), jnp.float32),
        jax.ShapeDtypeStruct((1, N_SITES), jnp.float32),
    ],
    scratch_types=[
        pltpu.VMEM((EPT,), jnp.int32),
        pltpu.VMEM((EPT,), jnp.int32),
        pltpu.VMEM((N,), jnp.int32),
        pltpu.VMEM((CAP,), jnp.int32),
        pltpu.VMEM((CAP,), jnp.int32),
        pltpu.VMEM((2, 128), jnp.int32),
        pltpu.VMEM((128, D), jnp.float32),
        pltpu.VMEM((128, D), jnp.float32),
        pltpu.VMEM((SPT,), jnp.int32),
        pltpu.VMEM((N,), jnp.float32),
        pltpu.VMEM((SPT,), jnp.float32),
        pltpu.VMEM_SHARED((SLOT_PAD, D), jnp.float32),
        pltpu.SemaphoreType.DMA,
        pltpu.SemaphoreType.DMA,
        pltpu.SemaphoreType.DMA,
        pltpu.SemaphoreType.DMA,
        pltpu.SemaphoreType.DMA,
    ],
    compiler_params=pltpu.CompilerParams(needs_layout_passes=False),
)
def _sc_edges(src_hbm, dst_hbm, pos_hbm, ni_hbm, dinv_hbm, y_hbm,
              hacc_out, dsites_out,
              src_v, dst_v, pos_v, csrc_v, cpos_v, idx2_v, rows0_v, rows1_v,
              ni_v, dinv_v, dsites_v, hacc_sh,
              sem_s, sem_d, sem_p, gsem0, gsem1):
    c = lax.axis_index("c")
    s = lax.axis_index("s")
    wid = s * NC + c
    iota = _iota16()
    zeros = jnp.zeros((L,), jnp.float32)

    base = wid * EPT
    d_src = pltpu.async_copy(src_hbm.at[pl.ds(base, EPT)], src_v, sem_s)
    d_dst = pltpu.async_copy(dst_hbm.at[pl.ds(base, EPT)], dst_v, sem_d)
    d_pos = pltpu.async_copy(pos_hbm, pos_v, sem_p)

    @pl.when(s == 0)
    def _zero_acc():
        def _zr(i, _):
            def _zc(j, _):
                rows0_v[i, pl.ds(j * L, L)] = zeros
                return 0
            lax.fori_loop(0, D // L, _zc, 0)
            return 0
        lax.fori_loop(0, 128, _zr, 0)

        def _zs(k, _):
            pltpu.sync_copy(rows0_v, hacc_sh.at[pl.ds(k * 128, 128)])
            return 0
        lax.fori_loop(0, SLOT_PAD // 128, _zs, 0)

    plsc.subcore_barrier()
    d_src.wait()
    d_dst.wait()
    d_pos.wait()

    def _filt(k, off):
        sv = src_v[pl.ds(k * L, L)]
        dv = dst_v[pl.ds(k * L, L)]
        p = plsc.load_gather(pos_v, [dv])
        m = p >= 0
        plsc.store_compressed(csrc_v.at[pl.ds(off, L)], sv, mask=m)
        plsc.store_compressed(cpos_v.at[pl.ds(off, L)], p, mask=m)
        return off + jnp.sum(jnp.where(m, 1, 0))
    off = lax.fori_loop(0, EPT // L, _filt, jnp.int32(0))

    def _with_selfloops(off):
        sbase = s * SPT
        d_ni = pltpu.async_copy(ni_hbm.at[pl.ds(sbase, SPT)], ni_v, sem_s)
        d_di = pltpu.async_copy(dinv_hbm.at[0], dinv_v, sem_d)
        d_ni.wait()
        d_di.wait()

        def _loop(k, off):
            nvec = ni_v[pl.ds(k * L, L)]
            pv = plsc.load_gather(pos_v, [nvec])
            m = pv == iota + (sbase + k * L)
            plsc.store_compressed(csrc_v.at[pl.ds(off, L)], nvec, mask=m)
            plsc.store_compressed(cpos_v.at[pl.ds(off, L)], pv, mask=m)
            dsites_v[pl.ds(k * L, L)] = plsc.load_gather(dinv_v, [nvec])
            return off + jnp.sum(jnp.where(m, 1, 0))
        off = lax.fori_loop(0, SPT // L, _loop, off)
        pltpu.sync_copy(dsites_v, dsites_out.at[0, pl.ds(sbase, SPT)])
        return off

    off = lax.cond(c == 0, _with_selfloops, lambda o: o, off)

    pad_src = jnp.zeros((L,), jnp.int32)
    pad_pos = jnp.full((L,), DUMMY, jnp.int32)

    def _pad(t, _):
        csrc_v[pl.ds(off + t * L, L)] = pad_src
        cpos_v[pl.ds(off + t * L, L)] = pad_pos
        return 0
    lax.fori_loop(0, 256 // L, _pad, 0)

    npairs = (off + 255) // 256

    def _fill_idx(b, j):
        def _mv(t, _):
            idx2_v[b, pl.ds(t * L, L)] = cpos_v[pl.ds(j * 128 + t * L, L)]
            return 0
        lax.fori_loop(0, 128 // L, _mv, 0)

    def _gather(j, rows, gsem):
        return pltpu.async_copy(y_hbm.at[csrc_v.at[pl.ds(j * 128, 128)]],
                                rows, gsem)

    def _drain(rows, gsem):
        pltpu.make_async_copy(y_hbm.at[pl.ds(0, 128)], rows, gsem).wait()

    @pl.when(npairs > 0)
    def _prime():
        _gather(0, rows0_v, gsem0)

    def _pair(g, _):
        j0 = 2 * g
        _drain(rows0_v, gsem0)
        _gather(j0 + 1, rows1_v, gsem1)
        _fill_idx(0, j0)
        pltpu.sync_copy(rows0_v, hacc_sh.at[idx2_v.at[0]], add=True)
        _drain(rows1_v, gsem1)

        @pl.when(g + 1 < npairs)
        def _next():
            _gather(j0 + 2, rows0_v, gsem0)
        _fill_idx(1, j0 + 1)
        pltpu.sync_copy(rows1_v, hacc_sh.at[idx2_v.at[1]], add=True)
        return 0
    lax.fori_loop(0, npairs, _pair, 0)

    plsc.subcore_barrier()

    @pl.when(s == 0)
    def _flush():
        def _cp(k, _):
            pltpu.sync_copy(hacc_sh.at[pl.ds(k * 128, 128)],
                            hacc_out.at[c, pl.ds(k * 128, 128)])
            return 0
        lax.fori_loop(0, SLOT_PAD // 128, _cp, 0)


def _leaky(v):
    return jnp.where(v > 0, v, 0.01 * v)


def _tc_head_body(hacc_ref, ps_ref, ds_ref, bconv_ref,
                  wpro_ref, bpro_ref, w1_ref, b1_ref, w2_ref, b2_ref,
                  wout_ref, bout_ref, out_ref):
    hacc = hacc_ref[0, :N_SITES, :] + hacc_ref[1, :N_SITES, :]
    slot_iota = lax.broadcasted_iota(jnp.int32, (N_SITES, N_SITES), 0)
    pt = jnp.where(slot_iota == ps_ref[...], ds_ref[...], 0.0)
    h = lax.dot_general(pt, hacc, (((0,), (0,)), ((), ())),
                        preferred_element_type=jnp.float32)
    h = _leaky(h + bconv_ref[...])
    h = _leaky(jnp.dot(h, wpro_ref[...], preferred_element_type=jnp.float32)
               + bpro_ref[...])
    h = _leaky(jnp.dot(h, w1_ref[...], preferred_element_type=jnp.float32)
               + b1_ref[...])
    h = _leaky(jnp.dot(h, w2_ref[...], preferred_element_type=jnp.float32)
               + b2_ref[...])
    logits = jnp.dot(h, wout_ref[...], preferred_element_type=jnp.float32) \
        + bout_ref[...]
    m = jnp.max(logits, axis=1, keepdims=True)
    e = jnp.exp(logits - m)
    out_ref[...] = e / jnp.sum(e, axis=1, keepdims=True)


def _tc_head(hacc, ps, dsites, b_conv, w_pro, b_pro, w1, b1, w2, b2,
             w_out, b_out):
    return pl.pallas_call(
        _tc_head_body,
        out_shape=jax.ShapeDtypeStruct((N_SITES, 10), jnp.float32),
    )(hacc, ps, dsites, b_conv[None, :], w_pro, b_pro[None, :],
      w1, b1[None, :], w2, b2[None, :], w_out, b_out[None, :])


def kernel(x, edge_index, node_index, W_conv, b_conv, W_pro, b_pro,
           W1, b1, W2, b2, W_out, b_out):
    src = edge_index[0]
    dst = edge_index[1]
    deg, pos, ps = _sc_deg_pos(dst, node_index)
    y, dinv = _tc_y(x, W_conv, deg)
    hacc, dsites = _sc_edges(src, dst, pos, node_index, dinv, y)
    return _tc_head(hacc, ps, dsites, b_conv, W_pro, b_pro,
                    W1, b1, W2, b2, W_out, b_out)


# speedup vs baseline: 1.0207x; 1.0207x over previous
import functools

import jax
import jax.numpy as jnp
from jax import lax
from jax.experimental import pallas as pl
from jax.experimental.pallas import tpu as pltpu
from jax.experimental.pallas import tpu_sc as plsc

N = 10000
E = 320000
D = 128
N_SITES = 1024
NC = 2
NSUB = 16
NW = NC * NSUB
EPT = E // NW
L = 16

SLOT_PAD = 1152
DUMMY = N_SITES
SPT = N_SITES // NSUB
CAP = EPT + SPT + 512

_mesh = plsc.VectorSubcoreMesh(core_axis_name="c", subcore_axis_name="s")


def _iota16():
    return lax.broadcasted_iota(jnp.int32, (L,), 0)


@functools.partial(
    pl.kernel,
    mesh=_mesh,
    out_type=[
        jax.ShapeDtypeStruct((NW, N), jnp.float32),
        jax.ShapeDtypeStruct((N,), jnp.int32),
        jax.ShapeDtypeStruct((1, N_SITES), jnp.int32),
        jax.ShapeDtypeStruct((NW, N_SITES), jnp.float32),
    ],
    scratch_types=[
        pltpu.VMEM((EPT,), jnp.int32),
        pltpu.VMEM((N,), jnp.float32),
        pltpu.VMEM((N,), jnp.int32),
        pltpu.VMEM((N_SITES,), jnp.int32),
        pltpu.VMEM((N_SITES,), jnp.int32),
        pltpu.VMEM((N_SITES,), jnp.float32),
        pltpu.SemaphoreType.DMA,
        pltpu.SemaphoreType.DMA,
    ],
    compiler_params=pltpu.CompilerParams(needs_layout_passes=False),
)
def _sc_deg_pos(dst_hbm, ni_hbm, deg_out, pos_out, ps_out, hsites_out,
                dst_v, hist_v, pos_v, ni_v, ps_v, hs_v, sem_d, sem_n):
    c = lax.axis_index("c")
    s = lax.axis_index("s")
    wid = s * NC + c
    ones = jnp.ones((L,), jnp.float32)
    zeros = jnp.zeros((L,), jnp.float32)
    iota = _iota16()

    d_dst = pltpu.async_copy(dst_hbm.at[pl.ds(wid * EPT, EPT)], dst_v, sem_d)
    d_ni = pltpu.async_copy(ni_hbm, ni_v, sem_n)

    def _zero(k, _):
        hist_v[pl.ds(k * L, L)] = zeros
        return 0
    lax.fori_loop(0, N // L, _zero, 0)
    d_dst.wait()

    def _hist(k, _):
        d = dst_v[pl.ds(k * L, L)]
        plsc.addupdate_scatter(hist_v, [d], ones)
        return 0
    lax.fori_loop(0, EPT // L, _hist, 0)

    pltpu.sync_copy(hist_v, deg_out.at[wid])

    d_ni.wait()

    def _hgath(k, _):
        idx = ni_v[pl.ds(k * L, L)]
        hs_v[pl.ds(k * L, L)] = plsc.load_gather(hist_v, [idx])
        return 0
    lax.fori_loop(0, N_SITES // L, _hgath, 0)
    pltpu.sync_copy(hs_v, hsites_out.at[wid])

    @pl.when(jnp.logical_and(c == 0, s == 0))
    def _tile0():
        neg = jnp.full((L,), -1, jnp.int32)

        def _init(k, _):
            pos_v[pl.ds(k * L, L)] = neg
            return 0
        lax.fori_loop(0, N // L, _init, 0)

        def _scat(k, _):
            idx = ni_v[pl.ds(k * L, L)]
            plsc.store_scatter(pos_v, [idx], iota + k * L)
            return 0
        lax.fori_loop(0, N_SITES // L, _scat, 0)

        def _gath(k, _):
            idx = ni_v[pl.ds(k * L, L)]
            ps_v[pl.ds(k * L, L)] = plsc.load_gather(pos_v, [idx])
            return 0
        lax.fori_loop(0, N_SITES // L, _gath, 0)

        pltpu.sync_copy(pos_v, pos_out)
        pltpu.sync_copy(ps_v, ps_out.at[0])


def _tc_y_body(x_ref, w_ref, deg_ref, y_ref):
    deg = jnp.sum(deg_ref[...], axis=0) + 1.0
    dinv = lax.rsqrt(deg)
    xw = jnp.dot(x_ref[...], w_ref[...], preferred_element_type=jnp.float32)
    y_ref[...] = xw * dinv[:, None]


def _tc_y(x, w_conv, deg):
    return pl.pallas_call(
        _tc_y_body,
        out_shape=jax.ShapeDtypeStruct((N, D), jnp.float32),
    )(x, w_conv, deg)


@functools.partial(
    pl.kernel,
    mesh=_mesh,
    out_type=[
        jax.ShapeDtypeStruct((NC, SLOT_PAD, D), jnp.float32),
    ],
    scratch_types=[
        pltpu.VMEM((EPT,), jnp.int32),
        pltpu.VMEM((EPT,), jnp.int32),
        pltpu.VMEM((N,), jnp.int32),
        pltpu.VMEM((CAP,), jnp.int32),
        pltpu.VMEM((CAP,), jnp.int32),
        pltpu.VMEM((2, 128), jnp.int32),
        pltpu.VMEM((128, D), jnp.float32),
        pltpu.VMEM((128, D), jnp.float32),
        pltpu.VMEM((SPT,), jnp.int32),
        pltpu.VMEM_SHARED((SLOT_PAD, D), jnp.float32),
        pltpu.SemaphoreType.DMA,
        pltpu.SemaphoreType.DMA,
        pltpu.SemaphoreType.DMA,
        pltpu.SemaphoreType.DMA,
        pltpu.SemaphoreType.DMA,
    ],
    compiler_params=pltpu.CompilerParams(needs_layout_passes=False),
)
def _sc_edges(src_hbm, dst_hbm, pos_hbm, ni_hbm, y_hbm,
              hacc_out,
              src_v, dst_v, pos_v, csrc_v, cpos_v, idx2_v, rows0_v, rows1_v,
              ni_v, hacc_sh,
              sem_s, sem_d, sem_p, gsem0, gsem1):
    c = lax.axis_index("c")
    s = lax.axis_index("s")
    wid = s * NC + c
    iota = _iota16()
    zeros = jnp.zeros((L,), jnp.float32)

    base = wid * EPT
    d_src = pltpu.async_copy(src_hbm.at[pl.ds(base, EPT)], src_v, sem_s)
    d_dst = pltpu.async_copy(dst_hbm.at[pl.ds(base, EPT)], dst_v, sem_d)
    d_pos = pltpu.async_copy(pos_hbm, pos_v, sem_p)

    ZR = SLOT_PAD // NSUB

    def _zr(i, _):
        def _zc(j, _):
            rows0_v[i, pl.ds(j * L, L)] = zeros
            return 0
        lax.fori_loop(0, D // L, _zc, 0)
        return 0
    lax.fori_loop(0, ZR, _zr, 0)
    pltpu.sync_copy(rows0_v.at[pl.ds(0, ZR)], hacc_sh.at[pl.ds(s * ZR, ZR)])

    plsc.subcore_barrier()
    d_src.wait()
    d_dst.wait()
    d_pos.wait()

    def _filt(k, off):
        sv = src_v[pl.ds(k * L, L)]
        dv = dst_v[pl.ds(k * L, L)]
        p = plsc.load_gather(pos_v, [dv])
        m = p >= 0
        plsc.store_compressed(csrc_v.at[pl.ds(off, L)], sv, mask=m)
        plsc.store_compressed(cpos_v.at[pl.ds(off, L)], p, mask=m)
        return off + jnp.sum(jnp.where(m, 1, 0))
    off = lax.fori_loop(0, EPT // L, _filt, jnp.int32(0))

    def _with_selfloops(off):
        sbase = s * SPT
        pltpu.sync_copy(ni_hbm.at[pl.ds(sbase, SPT)], ni_v)

        def _loop(k, off):
            nvec = ni_v[pl.ds(k * L, L)]
            pv = plsc.load_gather(pos_v, [nvec])
            m = pv == iota + (sbase + k * L)
            plsc.store_compressed(csrc_v.at[pl.ds(off, L)], nvec, mask=m)
            plsc.store_compressed(cpos_v.at[pl.ds(off, L)], pv, mask=m)
            return off + jnp.sum(jnp.where(m, 1, 0))
        off = lax.fori_loop(0, SPT // L, _loop, off)
        return off

    off = lax.cond(c == 0, _with_selfloops, lambda o: o, off)

    pad_src = jnp.zeros((L,), jnp.int32)
    pad_pos = jnp.full((L,), DUMMY, jnp.int32)

    def _pad(t, _):
        csrc_v[pl.ds(off + t * L, L)] = pad_src
        cpos_v[pl.ds(off + t * L, L)] = pad_pos
        return 0
    lax.fori_loop(0, 256 // L, _pad, 0)

    npairs = (off + 255) // 256

    def _fill_idx(b, j):
        def _mv(t, _):
            idx2_v[b, pl.ds(t * L, L)] = cpos_v[pl.ds(j * 128 + t * L, L)]
            return 0
        lax.fori_loop(0, 128 // L, _mv, 0)

    def _gather(j, rows, gsem):
        return pltpu.async_copy(y_hbm.at[csrc_v.at[pl.ds(j * 128, 128)]],
                                rows, gsem)

    def _drain(rows, gsem):
        pltpu.make_async_copy(y_hbm.at[pl.ds(0, 128)], rows, gsem).wait()

    @pl.when(npairs > 0)
    def _prime():
        _gather(0, rows0_v, gsem0)

    def _pair(g, _):
        j0 = 2 * g
        _drain(rows0_v, gsem0)
        _gather(j0 + 1, rows1_v, gsem1)
        _fill_idx(0, j0)
        pltpu.sync_copy(rows0_v, hacc_sh.at[idx2_v.at[0]], add=True)
        _drain(rows1_v, gsem1)

        @pl.when(g + 1 < npairs)
        def _next():
            _gather(j0 + 2, rows0_v, gsem0)
        _fill_idx(1, j0 + 1)
        pltpu.sync_copy(rows1_v, hacc_sh.at[idx2_v.at[1]], add=True)
        return 0
    lax.fori_loop(0, npairs, _pair, 0)

    plsc.subcore_barrier()

    @pl.when(s == 0)
    def _flush():
        def _cp(k, _):
            pltpu.sync_copy(hacc_sh.at[pl.ds(k * 128, 128)],
                            hacc_out.at[c, pl.ds(k * 128, 128)])
            return 0
        lax.fori_loop(0, SLOT_PAD // 128, _cp, 0)


def _leaky(v):
    return jnp.where(v > 0, v, 0.01 * v)


def _tc_head_body(hacc_ref, ps_ref, hs_ref, bconv_ref,
                  wpro_ref, bpro_ref, w1_ref, b1_ref, w2_ref, b2_ref,
                  wout_ref, bout_ref, out_ref):
    hacc = hacc_ref[0, :N_SITES, :] + hacc_ref[1, :N_SITES, :]
    dinv_sites = lax.rsqrt(jnp.sum(hs_ref[...], axis=0, keepdims=True) + 1.0)
    slot_iota = lax.broadcasted_iota(jnp.int32, (N_SITES, N_SITES), 0)
    pt = jnp.where(slot_iota == ps_ref[...], dinv_sites, 0.0)
    h = lax.dot_general(pt, hacc, (((0,), (0,)), ((), ())),
                        preferred_element_type=jnp.float32)
    h = _leaky(h + bconv_ref[...])
    h = _leaky(jnp.dot(h, wpro_ref[...], preferred_element_type=jnp.float32)
               + bpro_ref[...])
    h = _leaky(jnp.dot(h, w1_ref[...], preferred_element_type=jnp.float32)
               + b1_ref[...])
    h = _leaky(jnp.dot(h, w2_ref[...], preferred_element_type=jnp.float32)
               + b2_ref[...])
    logits = jnp.dot(h, wout_ref[...], preferred_element_type=jnp.float32) \
        + bout_ref[...]
    m = jnp.max(logits, axis=1, keepdims=True)
    e = jnp.exp(logits - m)
    out_ref[...] = e / jnp.sum(e, axis=1, keepdims=True)


def _tc_head(hacc, ps, hsites, b_conv, w_pro, b_pro, w1, b1, w2, b2,
             w_out, b_out):
    return pl.pallas_call(
        _tc_head_body,
        out_shape=jax.ShapeDtypeStruct((N_SITES, 10), jnp.float32),
    )(hacc, ps, hsites, b_conv[None, :], w_pro, b_pro[None, :],
      w1, b1[None, :], w2, b2[None, :], w_out, b_out[None, :])


def kernel(x, edge_index, node_index, W_conv, b_conv, W_pro, b_pro,
           W1, b1, W2, b2, W_out, b_out):
    src = edge_index[0]
    dst = edge_index[1]
    deg, pos, ps, hsites = _sc_deg_pos(dst, node_index)
    y = _tc_y(x, W_conv, deg)
    (hacc,) = _sc_edges(src, dst, pos, node_index, y)
    return _tc_head(hacc, ps, hsites, b_conv, W_pro, b_pro,
                    W1, b1, W2, b2, W_out, b_out)


# speedup vs baseline: 1.3905x; 1.3623x over previous
import functools

import jax
import jax.numpy as jnp
from jax import lax
from jax.experimental import pallas as pl
from jax.experimental.pallas import tpu as pltpu
from jax.experimental.pallas import tpu_sc as plsc

N = 10000
E = 320000
D = 128
N_SITES = 1024
NC = 2
NSUB = 16
NW = NC * NSUB
EPT = E // NW
L = 16

SLOT_PAD = 1152
DUMMY = N_SITES
YPT = N_SITES // NW
CAP = EPT + 512

_mesh = plsc.VectorSubcoreMesh(core_axis_name="c", subcore_axis_name="s")


def _iota16():
    return lax.broadcasted_iota(jnp.int32, (L,), 0)


@functools.partial(
    pl.kernel,
    mesh=_mesh,
    out_type=[
        jax.ShapeDtypeStruct((NW, N), jnp.float32),
        jax.ShapeDtypeStruct((N,), jnp.int32),
        jax.ShapeDtypeStruct((1, N_SITES), jnp.int32),
        jax.ShapeDtypeStruct((NW, N_SITES), jnp.float32),
    ],
    scratch_types=[
        pltpu.VMEM((EPT,), jnp.int32),
        pltpu.VMEM((N,), jnp.float32),
        pltpu.VMEM((N,), jnp.int32),
        pltpu.VMEM((N_SITES,), jnp.int32),
        pltpu.VMEM((N_SITES,), jnp.int32),
        pltpu.VMEM((N_SITES,), jnp.float32),
        pltpu.SemaphoreType.DMA,
        pltpu.SemaphoreType.DMA,
    ],
    compiler_params=pltpu.CompilerParams(needs_layout_passes=False),
)
def _sc_deg_pos(dst_hbm, ni_hbm, deg_out, pos_out, ps_out, hsites_out,
                dst_v, hist_v, pos_v, ni_v, ps_v, hs_v, sem_d, sem_n):
    c = lax.axis_index("c")
    s = lax.axis_index("s")
    wid = s * NC + c
    ones = jnp.ones((L,), jnp.float32)
    zeros = jnp.zeros((L,), jnp.float32)
    iota = _iota16()

    d_dst = pltpu.async_copy(dst_hbm.at[pl.ds(wid * EPT, EPT)], dst_v, sem_d)
    d_ni = pltpu.async_copy(ni_hbm, ni_v, sem_n)

    def _zero(k, _):
        hist_v[pl.ds(k * L, L)] = zeros
        return 0
    lax.fori_loop(0, N // L, _zero, 0)
    d_dst.wait()

    def _hist(k, _):
        d = dst_v[pl.ds(k * L, L)]
        plsc.addupdate_scatter(hist_v, [d], ones)
        return 0
    lax.fori_loop(0, EPT // L, _hist, 0)

    pltpu.sync_copy(hist_v, deg_out.at[wid])

    d_ni.wait()

    def _hgath(k, _):
        idx = ni_v[pl.ds(k * L, L)]
        hs_v[pl.ds(k * L, L)] = plsc.load_gather(hist_v, [idx])
        return 0
    lax.fori_loop(0, N_SITES // L, _hgath, 0)
    pltpu.sync_copy(hs_v, hsites_out.at[wid])

    @pl.when(jnp.logical_and(c == 0, s == 0))
    def _tile0():
        neg = jnp.full((L,), -1, jnp.int32)

        def _init(k, _):
            pos_v[pl.ds(k * L, L)] = neg
            return 0
        lax.fori_loop(0, N // L, _init, 0)

        def _scat(k, _):
            idx = ni_v[pl.ds(k * L, L)]
            plsc.store_scatter(pos_v, [idx], iota + k * L)
            return 0
        lax.fori_loop(0, N_SITES // L, _scat, 0)

        def _gath(k, _):
            idx = ni_v[pl.ds(k * L, L)]
            ps_v[pl.ds(k * L, L)] = plsc.load_gather(pos_v, [idx])
            return 0
        lax.fori_loop(0, N_SITES // L, _gath, 0)

        pltpu.sync_copy(pos_v, pos_out)
        pltpu.sync_copy(ps_v, ps_out.at[0])


def _tc_y_body(x_ref, w_ref, deg_ref, y_ref):
    deg = jnp.sum(deg_ref[...], axis=0) + 1.0
    dinv = lax.rsqrt(deg)
    xw = jnp.dot(x_ref[...], w_ref[...], preferred_element_type=jnp.float32)
    y_ref[...] = xw * dinv[:, None]


def _tc_y(x, w_conv, deg):
    return pl.pallas_call(
        _tc_y_body,
        out_shape=jax.ShapeDtypeStruct((N, D), jnp.float32),
    )(x, w_conv, deg)


@functools.partial(
    pl.kernel,
    mesh=_mesh,
    out_type=[
        jax.ShapeDtypeStruct((NC, SLOT_PAD, D), jnp.float32),
        jax.ShapeDtypeStruct((N_SITES, D), jnp.float32),
    ],
    scratch_types=[
        pltpu.VMEM((EPT,), jnp.int32),
        pltpu.VMEM((EPT,), jnp.int32),
        pltpu.VMEM((N,), jnp.int32),
        pltpu.VMEM((CAP,), jnp.int32),
        pltpu.VMEM((CAP,), jnp.int32),
        pltpu.VMEM((2, 128), jnp.int32),
        pltpu.VMEM((128, D), jnp.float32),
        pltpu.VMEM((128, D), jnp.float32),
        pltpu.VMEM((YPT,), jnp.int32),
        pltpu.VMEM((YPT, D), jnp.float32),
        pltpu.VMEM_SHARED((SLOT_PAD, D), jnp.float32),
        pltpu.SemaphoreType.DMA,
        pltpu.SemaphoreType.DMA,
        pltpu.SemaphoreType.DMA,
        pltpu.SemaphoreType.DMA,
        pltpu.SemaphoreType.DMA,
    ],
    compiler_params=pltpu.CompilerParams(needs_layout_passes=False),
)
def _sc_edges(src_hbm, dst_hbm, pos_hbm, ni_hbm, y_hbm,
              hacc_out, ysites_out,
              src_v, dst_v, pos_v, csrc_v, cpos_v, idx2_v, rows0_v, rows1_v,
              ni_v, yrow_v, hacc_sh,
              sem_s, sem_d, sem_p, gsem0, gsem1):
    c = lax.axis_index("c")
    s = lax.axis_index("s")
    wid = s * NC + c
    iota = _iota16()
    zeros = jnp.zeros((L,), jnp.float32)

    base = wid * EPT
    d_src = pltpu.async_copy(src_hbm.at[pl.ds(base, EPT)], src_v, sem_s)
    d_dst = pltpu.async_copy(dst_hbm.at[pl.ds(base, EPT)], dst_v, sem_d)
    d_pos = pltpu.async_copy(pos_hbm, pos_v, sem_p)

    ZR = SLOT_PAD // NSUB

    def _zr(i, _):
        def _zc(j, _):
            rows0_v[i, pl.ds(j * L, L)] = zeros
            return 0
        lax.fori_loop(0, D // L, _zc, 0)
        return 0
    lax.fori_loop(0, ZR, _zr, 0)
    pltpu.sync_copy(rows0_v.at[pl.ds(0, ZR)], hacc_sh.at[pl.ds(s * ZR, ZR)])

    plsc.subcore_barrier()
    d_src.wait()
    d_dst.wait()
    d_pos.wait()

    def _filt(k, off):
        sv = src_v[pl.ds(k * L, L)]
        dv = dst_v[pl.ds(k * L, L)]
        p = plsc.load_gather(pos_v, [dv])
        m = p >= 0
        plsc.store_compressed(csrc_v.at[pl.ds(off, L)], sv, mask=m)
        plsc.store_compressed(cpos_v.at[pl.ds(off, L)], p, mask=m)
        return off + jnp.sum(jnp.where(m, 1, 0))
    off = lax.fori_loop(0, EPT // L, _filt, jnp.int32(0))

    pad_src = jnp.zeros((L,), jnp.int32)
    pad_pos = jnp.full((L,), DUMMY, jnp.int32)

    def _pad(t, _):
        csrc_v[pl.ds(off + t * L, L)] = pad_src
        cpos_v[pl.ds(off + t * L, L)] = pad_pos
        return 0
    lax.fori_loop(0, 256 // L, _pad, 0)

    npairs = (off + 255) // 256

    def _fill_idx(b, j):
        def _mv(t, _):
            idx2_v[b, pl.ds(t * L, L)] = cpos_v[pl.ds(j * 128 + t * L, L)]
            return 0
        lax.fori_loop(0, 128 // L, _mv, 0)

    def _gather(j, rows, gsem):
        return pltpu.async_copy(y_hbm.at[csrc_v.at[pl.ds(j * 128, 128)]],
                                rows, gsem)

    def _drain(rows, gsem):
        pltpu.make_async_copy(y_hbm.at[pl.ds(0, 128)], rows, gsem).wait()

    @pl.when(npairs > 0)
    def _prime():
        _gather(0, rows0_v, gsem0)

    def _pair(g, _):
        j0 = 2 * g
        _drain(rows0_v, gsem0)
        _gather(j0 + 1, rows1_v, gsem1)
        _fill_idx(0, j0)
        pltpu.sync_copy(rows0_v, hacc_sh.at[idx2_v.at[0]], add=True)
        _drain(rows1_v, gsem1)

        @pl.when(g + 1 < npairs)
        def _next():
            _gather(j0 + 2, rows0_v, gsem0)
        _fill_idx(1, j0 + 1)
        pltpu.sync_copy(rows1_v, hacc_sh.at[idx2_v.at[1]], add=True)
        return 0
    lax.fori_loop(0, npairs, _pair, 0)

    ybase = wid * YPT
    pltpu.sync_copy(ni_hbm.at[pl.ds(ybase, YPT)], ni_v)
    pltpu.async_copy(y_hbm.at[ni_v], yrow_v, sem_p).wait()
    pltpu.sync_copy(yrow_v, ysites_out.at[pl.ds(ybase, YPT)])

    plsc.subcore_barrier()

    pltpu.sync_copy(hacc_sh.at[pl.ds(s * ZR, ZR)],
                    hacc_out.at[c, pl.ds(s * ZR, ZR)])


def _leaky(v):
    return jnp.where(v > 0, v, 0.01 * v)


def _tc_head_body(hacc_ref, ysites_ref, ps_ref, hs_ref, bconv_ref,
                  wpro_ref, bpro_ref, w1_ref, b1_ref, w2_ref, b2_ref,
                  wout_ref, bout_ref, out_ref):
    hacc = hacc_ref[0, :N_SITES, :] + hacc_ref[1, :N_SITES, :]
    ones_col = jnp.ones((NW, 1), jnp.float32)
    deg_col = lax.dot_general(hs_ref[...], ones_col, (((0,), (0,)), ((), ())),
                              preferred_element_type=jnp.float32)
    dinv_col = lax.rsqrt(deg_col + 1.0)
    slot_iota = lax.broadcasted_iota(jnp.int32, (N_SITES, N_SITES), 0)
    pt = jnp.where(slot_iota == ps_ref[...], 1.0, 0.0)
    h = lax.dot_general(pt, hacc, (((0,), (0,)), ((), ())),
                        preferred_element_type=jnp.float32)
    h = (h + ysites_ref[...]) * dinv_col
    h = _leaky(h + bconv_ref[...])
    h = _leaky(jnp.dot(h, wpro_ref[...], preferred_element_type=jnp.float32)
               + bpro_ref[...])
    h = _leaky(jnp.dot(h, w1_ref[...], preferred_element_type=jnp.float32)
               + b1_ref[...])
    h = _leaky(jnp.dot(h, w2_ref[...], preferred_element_type=jnp.float32)
               + b2_ref[...])
    logits = jnp.dot(h, wout_ref[...], preferred_element_type=jnp.float32) \
        + bout_ref[...]
    m = jnp.max(logits, axis=1, keepdims=True)
    e = jnp.exp(logits - m)
    out_ref[...] = e / jnp.sum(e, axis=1, keepdims=True)


def _tc_head(hacc, ysites, ps, hsites, b_conv, w_pro, b_pro, w1, b1, w2, b2,
             w_out, b_out):
    return pl.pallas_call(
        _tc_head_body,
        out_shape=jax.ShapeDtypeStruct((N_SITES, 10), jnp.float32),
    )(hacc, ysites, ps, hsites, b_conv[None, :], w_pro, b_pro[None, :],
      w1, b1[None, :], w2, b2[None, :], w_out, b_out[None, :])


def kernel(x, edge_index, node_index, W_conv, b_conv, W_pro, b_pro,
           W1, b1, W2, b2, W_out, b_out):
    src = edge_index[0]
    dst = edge_index[1]
    deg, pos, ps, hsites = _sc_deg_pos(dst, node_index)
    y = _tc_y(x, W_conv, deg)
    hacc, ysites = _sc_edges(src, dst, pos, node_index, y)
    return _tc_head(hacc, ysites, ps, hsites, b_conv, W_pro, b_pro,
                    W1, b1, W2, b2, W_out, b_out)
